# baseline (speedup 1.0000x reference)
import functools

import jax
import jax.numpy as jnp
from jax import lax
from jax.experimental import pallas as pl
from jax.experimental.pallas import tpu as pltpu
from jax.experimental.pallas import tpu_sc as plsc

T = 2048
D = 768
F = 2048
E = 8
K = 2
CAP = 256
A = T * K

_NEG_INF = float("-inf")


def _routing_kernel(logits_ref, e0_ref, e1_ref, d_ref, p0_ref, p1_ref,
                    colsum_ref, z_ref):
    logits = logits_ref[...]
    ex_iota = jax.lax.broadcasted_iota(jnp.int32, (T, E), 1)

    l0 = jnp.max(logits, axis=1, keepdims=True)
    is0 = logits == l0
    e0 = jnp.min(jnp.where(is0, ex_iota, E), axis=1, keepdims=True)
    masked = jnp.where(ex_iota == e0, _NEG_INF, logits)
    l1 = jnp.max(masked, axis=1, keepdims=True)
    is1 = masked == l1
    e1 = jnp.min(jnp.where(is1, ex_iota, E), axis=1, keepdims=True)

    u1 = jnp.exp(l1 - l0)
    den = 1.0 + u1
    p0u = 1.0 / den
    p1u = u1 / den
    s = jnp.maximum(p0u + p1u, 1e-8)
    p0_ref[...] = p0u / s
    p1_ref[...] = p1u / s

    e0_ref[...] = e0
    e1_ref[...] = e1
    d_ref[...] = l0 - l1

    exps = jnp.exp(logits - l0)
    den8 = jnp.sum(exps, axis=1, keepdims=True)
    colsum_ref[...] = jnp.sum(exps / den8, axis=0, keepdims=True)

    lse = l0 + jnp.log(den8)
    z_ref[...] = jnp.sum(lse * lse, axis=0, keepdims=True) * (0.001 / T)


_BI = 128


def _rank_kernel(scol_ref, ecol_ref, srow_ref, erow_ref,
                 rank_ref, keep_ref, dst_ref, src_ref):
    i = pl.program_id(0)
    scol = scol_ref[...]
    ecol = ecol_ref[...]
    srow = srow_ref[...]
    erow = erow_ref[...]
    icol = i * _BI + jax.lax.broadcasted_iota(jnp.int32, (_BI, 1), 0)
    irow = jax.lax.broadcasted_iota(jnp.int32, (1, A), 1)

    same_e = erow == ecol
    beats = (srow > scol) | ((srow == scol) & (irow < icol))
    cnt = jnp.sum(jnp.where(same_e & beats, 1.0, 0.0), axis=1, keepdims=True)
    rank = cnt.astype(jnp.int32)
    keep = rank < CAP
    rank_ref[...] = rank
    keep_ref[...] = keep.astype(jnp.int32)
    row = ecol * CAP + rank
    dst_ref[...] = jnp.where(keep, row, E * CAP + (icol & (CAP - 1)))
    src_ref[...] = jnp.where(keep, row, E * CAP)


_FB = 2
_FBLK = F // _FB
_NW = 32
_TPW = T // _NW


def _expert_kernel(buf_ref, w1_ref, w3_ref, w2_ref,
                   e0c_ref, e1c_ref, r0c_ref, r1c_ref, k0c_ref, k1c_ref,
                   p0c_ref, p1c_ref, colsum_ref,
                   yw_ref, lb_ref, y_ref):
    e = pl.program_id(0)
    fb = pl.program_id(1)

    @pl.when((e == 0) & (fb == 0))
    def _init():
        lb_ref[...] = jnp.zeros((1, 1), jnp.float32)

    @pl.when((e == E) & (fb == 0))
    def _zero_dummy():
        yw_ref[...] = jnp.zeros((CAP, D), jnp.float32)

    @pl.when(e < E)
    def _ffn():
        buf = buf_ref[...].astype(jnp.bfloat16)
        w1 = w1_ref[0].astype(jnp.bfloat16)
        w3 = w3_ref[0].astype(jnp.bfloat16)
        w2 = w2_ref[0].astype(jnp.bfloat16)
        h1 = jax.lax.dot_general(buf, w1, (((1,), (1,)), ((), ())),
                                 preferred_element_type=jnp.float32)
        h3 = jax.lax.dot_general(buf, w3, (((1,), (1,)), ((), ())),
                                 preferred_element_type=jnp.float32)
        h = h1 * jax.lax.logistic(h1) * h3
        y_part = jax.lax.dot_general(h.astype(jnp.bfloat16), w2,
                                     (((1,), (1,)), ((), ())),
                                     preferred_element_type=jnp.float32)

        @pl.when(fb == 0)
        def _y_init():
            y_ref[...] = y_part

        @pl.when(fb != 0)
        def _y_acc():
            y_ref[...] += y_part

        @pl.when(fb == _FB - 1)
        def _weight():
            q_col = jnp.where(
                (e0c_ref[...] == e) & (k0c_ref[...] == 1), r0c_ref[...],
                jnp.where((e1c_ref[...] == e) & (k1c_ref[...] == 1),
                          r1c_ref[...], -1))
            pe_col = jnp.where(
                (e0c_ref[...] == e) & (k0c_ref[...] == 1), p0c_ref[...],
                jnp.where((e1c_ref[...] == e) & (k1c_ref[...] == 1),
                          p1c_ref[...], 0.0))
            slot_iota = jax.lax.broadcasted_iota(jnp.int32, (T, CAP), 1)
            pt = (slot_iota == q_col).astype(jnp.float32)
            w_slot = jax.lax.dot_general(
                pt, pe_col, (((0,), (0,)), ((), ())))
            yw_ref[...] = y_ref[...] * w_slot

            cnt = jnp.sum(jnp.where((e0c_ref[...] == e) & (k0c_ref[...] == 1),
                                    1.0, 0.0), axis=0, keepdims=True)
            lane_iota = jax.lax.broadcasted_iota(jnp.int32, (1, E), 1)
            cs_e = jnp.sum(jnp.where(lane_iota == e, colsum_ref[...], 0.0),
                           axis=1, keepdims=True)
            lb_ref[...] += cs_e * cnt * (0.01 / (T * E))


_SC_MESH = plsc.VectorSubcoreMesh(core_axis_name="c", subcore_axis_name="s")


def _sc_dispatch_body(x_hbm, d0_hbm, d1_hbm, buf_hbm, xv, i0, i1, sem):
    wid = lax.axis_index("s") * 2 + lax.axis_index("c")
    base = wid * _TPW
    pltpu.sync_copy(x_hbm.at[pl.ds(base, _TPW)], xv)
    pltpu.sync_copy(d0_hbm.at[pl.ds(base, _TPW)], i0)
    pltpu.sync_copy(d1_hbm.at[pl.ds(base, _TPW)], i1)
    pltpu.async_copy(xv, buf_hbm.at[i0], sem).wait()
    pltpu.async_copy(xv, buf_hbm.at[i1], sem).wait()


def _sc_combine_body(yw_hbm, s0_hbm, s1_hbm, out_hbm, av, bv, i0, i1, sem):
    wid = lax.axis_index("s") * 2 + lax.axis_index("c")
    base = wid * _TPW
    pltpu.sync_copy(s0_hbm.at[pl.ds(base, _TPW)], i0)
    pltpu.sync_copy(s1_hbm.at[pl.ds(base, _TPW)], i1)
    pltpu.async_copy(yw_hbm.at[i0], av, sem).wait()
    pltpu.async_copy(yw_hbm.at[i1], bv, sem).wait()

    def add_row(r, _):
        for c in range(D // 16):
            sl = pl.ds(c * 16, 16)
            av[r, sl] = av[r, sl] + bv[r, sl]
        return 0

    lax.fori_loop(0, _TPW, add_row, 0)
    pltpu.sync_copy(av, out_hbm.at[pl.ds(base, _TPW)])


_sc_dispatch = pl.kernel(
    _sc_dispatch_body,
    out_type=jax.ShapeDtypeStruct(((E + 1) * CAP, D), jnp.float32),
    mesh=_SC_MESH,
    scratch_types=[
        pltpu.VMEM((_TPW, D), jnp.float32),
        pltpu.VMEM((_TPW,), jnp.int32),
        pltpu.VMEM((_TPW,), jnp.int32),
        pltpu.SemaphoreType.DMA,
    ],
)

_sc_combine = pl.kernel(
    _sc_combine_body,
    out_type=jax.ShapeDtypeStruct((T, D), jnp.float32),
    mesh=_SC_MESH,
    scratch_types=[
        pltpu.VMEM((_TPW, D), jnp.float32),
        pltpu.VMEM((_TPW, D), jnp.float32),
        pltpu.VMEM((_TPW,), jnp.int32),
        pltpu.VMEM((_TPW,), jnp.int32),
        pltpu.SemaphoreType.DMA,
    ],
)


@jax.jit
def kernel(x, Wg, W1, W3, W2):
    logits = x @ Wg.T

    e0, e1, d, p0, p1, colsum, z = pl.pallas_call(
        _routing_kernel,
        out_shape=(
            jax.ShapeDtypeStruct((T, 1), jnp.int32),
            jax.ShapeDtypeStruct((T, 1), jnp.int32),
            jax.ShapeDtypeStruct((T, 1), jnp.float32),
            jax.ShapeDtypeStruct((T, 1), jnp.float32),
            jax.ShapeDtypeStruct((T, 1), jnp.float32),
            jax.ShapeDtypeStruct((1, E), jnp.float32),
            jax.ShapeDtypeStruct((1, 1), jnp.float32),
        ),
    )(logits)

    s_flat = jnp.concatenate([d, -d], axis=1).reshape(A, 1)
    e_flat = jnp.concatenate([e0, e1], axis=1).reshape(A, 1)

    rank, keep, dst, src = pl.pallas_call(
        _rank_kernel,
        grid=(A // _BI,),
        in_specs=[
            pl.BlockSpec((_BI, 1), lambda i: (i, 0)),
            pl.BlockSpec((_BI, 1), lambda i: (i, 0)),
            pl.BlockSpec((1, A), lambda i: (0, 0)),
            pl.BlockSpec((1, A), lambda i: (0, 0)),
        ],
        out_shape=(
            jax.ShapeDtypeStruct((A, 1), jnp.int32),
            jax.ShapeDtypeStruct((A, 1), jnp.int32),
            jax.ShapeDtypeStruct((A, 1), jnp.int32),
            jax.ShapeDtypeStruct((A, 1), jnp.int32),
        ),
        out_specs=tuple(
            pl.BlockSpec((_BI, 1), lambda i: (i, 0)) for _ in range(4)),
    )(s_flat, e_flat, s_flat.reshape(1, A), e_flat.reshape(1, A))

    rank2 = rank.reshape(T, K)
    keep2 = keep.reshape(T, K)
    r0c, r1c = rank2[:, 0:1], rank2[:, 1:2]
    k0c, k1c = keep2[:, 0:1], keep2[:, 1:2]
    dst2 = dst.reshape(T, K)
    src2 = src.reshape(T, K)

    buf = _sc_dispatch(x, dst2[:, 0], dst2[:, 1])

    wmin = lambda e: jnp.minimum(e, E - 1)
    yw, lb = pl.pallas_call(
        _expert_kernel,
        grid=(E + 1, _FB),
        in_specs=[
            pl.BlockSpec((CAP, D), lambda e, fb: (e, 0)),
            pl.BlockSpec((1, _FBLK, D), lambda e, fb: (wmin(e), fb, 0)),
            pl.BlockSpec((1, _FBLK, D), lambda e, fb: (wmin(e), fb, 0)),
            pl.BlockSpec((1, D, _FBLK), lambda e, fb: (wmin(e), 0, fb)),
        ] + [pl.BlockSpec((T, 1), lambda e, fb: (0, 0))] * 8
          + [pl.BlockSpec((1, E), lambda e, fb: (0, 0))],
        out_shape=(
            jax.ShapeDtypeStruct(((E + 1) * CAP, D), jnp.float32),
            jax.ShapeDtypeStruct((1, 1), jnp.float32),
        ),
        out_specs=(
            pl.BlockSpec((CAP, D), lambda e, fb: (e, 0)),
            pl.BlockSpec((1, 1), lambda e, fb: (0, 0)),
        ),
        scratch_shapes=[
            pltpu.VMEM((CAP, D), jnp.float32),
        ],
    )(buf, W1, W3, W2,
      e0, e1, r0c, r1c, k0c, k1c, p0, p1, colsum)

    out = _sc_combine(yw, src2[:, 0], src2[:, 1])

    return out, lb.reshape(()), z.reshape(())


# speedup vs baseline: 1.5604x; 1.5604x over previous
import functools

import jax
import jax.numpy as jnp
from jax import lax
from jax.experimental import pallas as pl
from jax.experimental.pallas import tpu as pltpu
from jax.experimental.pallas import tpu_sc as plsc

T = 2048
D = 768
F = 2048
E = 8
K = 2
CAP = 256
A = T * K

_NEG_INF = float("-inf")


def _routing_kernel(logits_ref, e0_ref, e1_ref, d_ref, p0_ref, p1_ref,
                    colsum_ref, z_ref):
    logits = logits_ref[...]
    ex_iota = jax.lax.broadcasted_iota(jnp.int32, (T, E), 1)

    l0 = jnp.max(logits, axis=1, keepdims=True)
    is0 = logits == l0
    e0 = jnp.min(jnp.where(is0, ex_iota, E), axis=1, keepdims=True)
    masked = jnp.where(ex_iota == e0, _NEG_INF, logits)
    l1 = jnp.max(masked, axis=1, keepdims=True)
    is1 = masked == l1
    e1 = jnp.min(jnp.where(is1, ex_iota, E), axis=1, keepdims=True)

    u1 = jnp.exp(l1 - l0)
    den = 1.0 + u1
    p0u = 1.0 / den
    p1u = u1 / den
    s = jnp.maximum(p0u + p1u, 1e-8)
    p0_ref[...] = p0u / s
    p1_ref[...] = p1u / s

    e0_ref[...] = e0
    e1_ref[...] = e1
    d_ref[...] = l0 - l1

    exps = jnp.exp(logits - l0)
    den8 = jnp.sum(exps, axis=1, keepdims=True)
    colsum_ref[...] = jnp.sum(exps / den8, axis=0, keepdims=True)

    lse = l0 + jnp.log(den8)
    z_ref[...] = jnp.sum(lse * lse, axis=0, keepdims=True) * (0.001 / T)


_BI = 128


def _rank_kernel(scol_ref, ecol_ref, srow_ref, erow_ref,
                 rank_ref, keep_ref, dst_ref, src_ref):
    i = pl.program_id(0)
    scol = scol_ref[...]
    ecol = ecol_ref[...]
    srow = srow_ref[...]
    erow = erow_ref[...]
    icol = i * _BI + jax.lax.broadcasted_iota(jnp.int32, (_BI, 1), 0)
    irow = jax.lax.broadcasted_iota(jnp.int32, (1, A), 1)

    same_e = erow == ecol
    beats = (srow > scol) | ((srow == scol) & (irow < icol))
    cnt = jnp.sum(jnp.where(same_e & beats, 1.0, 0.0), axis=1, keepdims=True)
    rank = cnt.astype(jnp.int32)
    keep = rank < CAP
    rank_ref[...] = rank
    keep_ref[...] = keep.astype(jnp.int32)
    row = ecol * CAP + rank
    dst_ref[...] = jnp.where(keep, row, E * CAP + (icol & (CAP - 1)))
    src_ref[...] = jnp.where(keep, row, E * CAP)


_FB = 2
_FBLK = F // _FB
_NW = 32
_TPW = T // _NW


def _expert_kernel(buf_ref, w1_ref, w3_ref, w2_ref,
                   e0c_ref, e1c_ref, r0c_ref, r1c_ref, k0c_ref, k1c_ref,
                   p0c_ref, p1c_ref, colsum_ref,
                   out_ref, lb_ref, y_ref):
    e = pl.program_id(0)
    fb = pl.program_id(1)

    @pl.when((e == 0) & (fb == 0))
    def _init():
        out_ref[...] = jnp.zeros((T, D), jnp.float32)
        lb_ref[...] = jnp.zeros((1, 1), jnp.float32)

    buf = buf_ref[...].astype(jnp.bfloat16)
    w1 = w1_ref[0].astype(jnp.bfloat16)
    w3 = w3_ref[0].astype(jnp.bfloat16)
    w2 = w2_ref[0].astype(jnp.bfloat16)
    h1 = jax.lax.dot_general(buf, w1, (((1,), (1,)), ((), ())),
                             preferred_element_type=jnp.float32)
    h3 = jax.lax.dot_general(buf, w3, (((1,), (1,)), ((), ())),
                             preferred_element_type=jnp.float32)
    h = h1 * jax.lax.logistic(h1) * h3
    y_part = jax.lax.dot_general(h.astype(jnp.bfloat16), w2,
                                 (((1,), (1,)), ((), ())),
                                 preferred_element_type=jnp.float32)

    @pl.when(fb == 0)
    def _y_init():
        y_ref[...] = y_part

    @pl.when(fb != 0)
    def _y_acc():
        y_ref[...] += y_part

    @pl.when(fb == _FB - 1)
    def _combine():
        q_col = jnp.where(
            (e0c_ref[...] == e) & (k0c_ref[...] == 1), r0c_ref[...],
            jnp.where((e1c_ref[...] == e) & (k1c_ref[...] == 1),
                      r1c_ref[...], -1))
        w_col = jnp.where(
            (e0c_ref[...] == e) & (k0c_ref[...] == 1), p0c_ref[...],
            jnp.where((e1c_ref[...] == e) & (k1c_ref[...] == 1),
                      p1c_ref[...], 0.0))
        tok_iota = jax.lax.broadcasted_iota(jnp.int32, (T, CAP), 1)
        comb = (tok_iota == q_col).astype(jnp.bfloat16)
        contrib = jax.lax.dot_general(
            comb, y_ref[...].astype(jnp.bfloat16), (((1,), (0,)), ((), ())),
            preferred_element_type=jnp.float32)
        out_ref[...] += w_col * contrib

        cnt = jnp.sum(jnp.where((e0c_ref[...] == e) & (k0c_ref[...] == 1),
                                1.0, 0.0), axis=0, keepdims=True)
        lane_iota = jax.lax.broadcasted_iota(jnp.int32, (1, E), 1)
        cs_e = jnp.sum(jnp.where(lane_iota == e, colsum_ref[...], 0.0),
                       axis=1, keepdims=True)
        lb_ref[...] += cs_e * cnt * (0.01 / (T * E))


_SC_MESH = plsc.VectorSubcoreMesh(core_axis_name="c", subcore_axis_name="s")


def _sc_dispatch_body(x_hbm, d0_hbm, d1_hbm, buf_hbm, xv, i0, i1, sem):
    wid = lax.axis_index("s") * 2 + lax.axis_index("c")
    base = wid * _TPW
    pltpu.sync_copy(x_hbm.at[pl.ds(base, _TPW)], xv)
    pltpu.sync_copy(d0_hbm.at[pl.ds(base, _TPW)], i0)
    pltpu.sync_copy(d1_hbm.at[pl.ds(base, _TPW)], i1)
    pltpu.async_copy(xv, buf_hbm.at[i0], sem).wait()
    pltpu.async_copy(xv, buf_hbm.at[i1], sem).wait()


def _sc_combine_body(yw_hbm, s0_hbm, s1_hbm, out_hbm, av, bv, i0, i1, sem):
    wid = lax.axis_index("s") * 2 + lax.axis_index("c")
    base = wid * _TPW
    pltpu.sync_copy(s0_hbm.at[pl.ds(base, _TPW)], i0)
    pltpu.sync_copy(s1_hbm.at[pl.ds(base, _TPW)], i1)
    pltpu.async_copy(yw_hbm.at[i0], av, sem).wait()
    pltpu.async_copy(yw_hbm.at[i1], bv, sem).wait()

    def add_row(r, _):
        for c in range(D // 16):
            sl = pl.ds(c * 16, 16)
            av[r, sl] = av[r, sl] + bv[r, sl]
        return 0

    lax.fori_loop(0, _TPW, add_row, 0)
    pltpu.sync_copy(av, out_hbm.at[pl.ds(base, _TPW)])


_sc_dispatch = pl.kernel(
    _sc_dispatch_body,
    out_type=jax.ShapeDtypeStruct(((E + 1) * CAP, D), jnp.float32),
    mesh=_SC_MESH,
    scratch_types=[
        pltpu.VMEM((_TPW, D), jnp.float32),
        pltpu.VMEM((_TPW,), jnp.int32),
        pltpu.VMEM((_TPW,), jnp.int32),
        pltpu.SemaphoreType.DMA,
    ],
)

_sc_combine = pl.kernel(
    _sc_combine_body,
    out_type=jax.ShapeDtypeStruct((T, D), jnp.float32),
    mesh=_SC_MESH,
    scratch_types=[
        pltpu.VMEM((_TPW, D), jnp.float32),
        pltpu.VMEM((_TPW, D), jnp.float32),
        pltpu.VMEM((_TPW,), jnp.int32),
        pltpu.VMEM((_TPW,), jnp.int32),
        pltpu.SemaphoreType.DMA,
    ],
)


@jax.jit
def kernel(x, Wg, W1, W3, W2):
    logits = x @ Wg.T

    e0, e1, d, p0, p1, colsum, z = pl.pallas_call(
        _routing_kernel,
        out_shape=(
            jax.ShapeDtypeStruct((T, 1), jnp.int32),
            jax.ShapeDtypeStruct((T, 1), jnp.int32),
            jax.ShapeDtypeStruct((T, 1), jnp.float32),
            jax.ShapeDtypeStruct((T, 1), jnp.float32),
            jax.ShapeDtypeStruct((T, 1), jnp.float32),
            jax.ShapeDtypeStruct((1, E), jnp.float32),
            jax.ShapeDtypeStruct((1, 1), jnp.float32),
        ),
    )(logits)

    s_flat = jnp.concatenate([d, -d], axis=1).reshape(A, 1)
    e_flat = jnp.concatenate([e0, e1], axis=1).reshape(A, 1)

    rank, keep, dst, src = pl.pallas_call(
        _rank_kernel,
        grid=(A // _BI,),
        in_specs=[
            pl.BlockSpec((_BI, 1), lambda i: (i, 0)),
            pl.BlockSpec((_BI, 1), lambda i: (i, 0)),
            pl.BlockSpec((1, A), lambda i: (0, 0)),
            pl.BlockSpec((1, A), lambda i: (0, 0)),
        ],
        out_shape=(
            jax.ShapeDtypeStruct((A, 1), jnp.int32),
            jax.ShapeDtypeStruct((A, 1), jnp.int32),
            jax.ShapeDtypeStruct((A, 1), jnp.int32),
            jax.ShapeDtypeStruct((A, 1), jnp.int32),
        ),
        out_specs=tuple(
            pl.BlockSpec((_BI, 1), lambda i: (i, 0)) for _ in range(4)),
    )(s_flat, e_flat, s_flat.reshape(1, A), e_flat.reshape(1, A))

    rank2 = rank.reshape(T, K)
    keep2 = keep.reshape(T, K)
    r0c, r1c = rank2[:, 0:1], rank2[:, 1:2]
    k0c, k1c = keep2[:, 0:1], keep2[:, 1:2]
    dst2 = dst.reshape(T, K)
    src2 = src.reshape(T, K)

    buf = _sc_dispatch(x, dst2[:, 0], dst2[:, 1])

    out, lb = pl.pallas_call(
        _expert_kernel,
        grid=(E, _FB),
        in_specs=[
            pl.BlockSpec((CAP, D), lambda e, fb: (e, 0)),
            pl.BlockSpec((1, _FBLK, D), lambda e, fb: (e, fb, 0)),
            pl.BlockSpec((1, _FBLK, D), lambda e, fb: (e, fb, 0)),
            pl.BlockSpec((1, D, _FBLK), lambda e, fb: (e, 0, fb)),
        ] + [pl.BlockSpec((T, 1), lambda e, fb: (0, 0))] * 8
          + [pl.BlockSpec((1, E), lambda e, fb: (0, 0))],
        out_shape=(
            jax.ShapeDtypeStruct((T, D), jnp.float32),
            jax.ShapeDtypeStruct((1, 1), jnp.float32),
        ),
        out_specs=(
            pl.BlockSpec((T, D), lambda e, fb: (0, 0)),
            pl.BlockSpec((1, 1), lambda e, fb: (0, 0)),
        ),
        scratch_shapes=[
            pltpu.VMEM((CAP, D), jnp.float32),
        ],
    )(buf, W1, W3, W2,
      e0, e1, r0c, r1c, k0c, k1c, p0, p1, colsum)

    return out, lb.reshape(()), z.reshape(())


# speedup vs baseline: 1.5620x; 1.0010x over previous
import jax
import jax.numpy as jnp
from jax import lax
from jax.experimental import pallas as pl
from jax.experimental.pallas import tpu as pltpu
from jax.experimental.pallas import tpu_sc as plsc

T = 2048
D = 768
F = 2048
E = 8
K = 2
CAP = 256
A = T * K

_NEG_INF = float("-inf")


def _routing_kernel(logits_ref, e0_ref, e1_ref, d_ref, p0_ref, p1_ref,
                    colsum_ref, z_ref):
    logits = logits_ref[...]
    ex_iota = jax.lax.broadcasted_iota(jnp.int32, (T, E), 1)

    l0 = jnp.max(logits, axis=1, keepdims=True)
    is0 = logits == l0
    e0 = jnp.min(jnp.where(is0, ex_iota, E), axis=1, keepdims=True)
    masked = jnp.where(ex_iota == e0, _NEG_INF, logits)
    l1 = jnp.max(masked, axis=1, keepdims=True)
    is1 = masked == l1
    e1 = jnp.min(jnp.where(is1, ex_iota, E), axis=1, keepdims=True)

    u1 = jnp.exp(l1 - l0)
    den = 1.0 + u1
    p0u = 1.0 / den
    p1u = u1 / den
    s = jnp.maximum(p0u + p1u, 1e-8)
    p0_ref[...] = p0u / s
    p1_ref[...] = p1u / s

    e0_ref[...] = e0
    e1_ref[...] = e1
    d_ref[...] = l0 - l1

    exps = jnp.exp(logits - l0)
    den8 = jnp.sum(exps, axis=1, keepdims=True)
    colsum_ref[...] = jnp.sum(exps / den8, axis=0, keepdims=True)

    lse = l0 + jnp.log(den8)
    z_ref[...] = jnp.sum(lse * lse, axis=0, keepdims=True) * (0.001 / T)


_BI = 128


def _rank_kernel(scol_ref, ecol_ref, srow_ref, erow_ref,
                 rank_ref, keep_ref, dst_ref):
    i = pl.program_id(0)
    scol = scol_ref[...]
    ecol = ecol_ref[...]
    srow = srow_ref[...]
    erow = erow_ref[...]
    icol = i * _BI + jax.lax.broadcasted_iota(jnp.int32, (_BI, 1), 0)
    irow = jax.lax.broadcasted_iota(jnp.int32, (1, A), 1)

    same_e = erow == ecol
    beats = (srow > scol) | ((srow == scol) & (irow < icol))
    cnt = jnp.sum(jnp.where(same_e & beats, 1.0, 0.0), axis=1, keepdims=True)
    rank = cnt.astype(jnp.int32)
    keep = rank < CAP
    rank_ref[...] = rank
    keep_ref[...] = keep.astype(jnp.int32)
    row = ecol * CAP + rank
    dst_ref[...] = jnp.where(keep, row, E * CAP + (icol & (CAP - 1)))


_FB = 2
_FBLK = F // _FB
_NW = 32
_TPW = T // _NW


def _expert_kernel(buf_ref, w1_ref, w3_ref, w2_ref,
                   e0c_ref, e1c_ref, r0c_ref, r1c_ref, k0c_ref, k1c_ref,
                   p0c_ref, p1c_ref, colsum_ref,
                   out_ref, lb_ref, y_ref):
    e = pl.program_id(0)
    fb = pl.program_id(1)

    @pl.when((e == 0) & (fb == 0))
    def _init():
        out_ref[...] = jnp.zeros((T, D), jnp.float32)
        lb_ref[...] = jnp.zeros((1, 1), jnp.float32)

    buf = buf_ref[...].astype(jnp.bfloat16)
    w1 = w1_ref[0].astype(jnp.bfloat16)
    w3 = w3_ref[0].astype(jnp.bfloat16)
    w2 = w2_ref[0].astype(jnp.bfloat16)
    h1 = jax.lax.dot_general(buf, w1, (((1,), (1,)), ((), ())),
                             preferred_element_type=jnp.float32)
    h3 = jax.lax.dot_general(buf, w3, (((1,), (1,)), ((), ())),
                             preferred_element_type=jnp.float32)
    h = h1 * jax.lax.logistic(h1) * h3
    y_part = jax.lax.dot_general(h.astype(jnp.bfloat16), w2,
                                 (((1,), (1,)), ((), ())),
                                 preferred_element_type=jnp.float32)

    @pl.when(fb == 0)
    def _y_init():
        y_ref[...] = y_part

    @pl.when(fb != 0)
    def _y_acc():
        y_ref[...] += y_part

    @pl.when(fb == _FB - 1)
    def _combine():
        q_col = jnp.where(
            (e0c_ref[...] == e) & (k0c_ref[...] == 1), r0c_ref[...],
            jnp.where((e1c_ref[...] == e) & (k1c_ref[...] == 1),
                      r1c_ref[...], -1))
        w_col = jnp.where(
            (e0c_ref[...] == e) & (k0c_ref[...] == 1), p0c_ref[...],
            jnp.where((e1c_ref[...] == e) & (k1c_ref[...] == 1),
                      p1c_ref[...], 0.0))
        tok_iota = jax.lax.broadcasted_iota(jnp.int32, (T, CAP), 1)
        comb = (tok_iota == q_col).astype(jnp.bfloat16)
        contrib = jax.lax.dot_general(
            comb, y_ref[...].astype(jnp.bfloat16), (((1,), (0,)), ((), ())),
            preferred_element_type=jnp.float32)
        out_ref[...] += w_col * contrib

        cnt = jnp.sum(jnp.where((e0c_ref[...] == e) & (k0c_ref[...] == 1),
                                1.0, 0.0), axis=0, keepdims=True)
        lane_iota = jax.lax.broadcasted_iota(jnp.int32, (1, E), 1)
        cs_e = jnp.sum(jnp.where(lane_iota == e, colsum_ref[...], 0.0),
                       axis=1, keepdims=True)
        lb_ref[...] += cs_e * cnt * (0.01 / (T * E))


_SC_MESH = plsc.VectorSubcoreMesh(core_axis_name="c", subcore_axis_name="s")


def _sc_dispatch_body(x_hbm, d0_hbm, d1_hbm, buf_hbm, xv, i0, i1, sem):
    wid = lax.axis_index("s") * 2 + lax.axis_index("c")
    base = wid * _TPW
    pltpu.sync_copy(x_hbm.at[pl.ds(base, _TPW)], xv)
    pltpu.sync_copy(d0_hbm.at[pl.ds(base, _TPW)], i0)
    pltpu.sync_copy(d1_hbm.at[pl.ds(base, _TPW)], i1)
    pltpu.async_copy(xv, buf_hbm.at[i0], sem).wait()
    pltpu.async_copy(xv, buf_hbm.at[i1], sem).wait()


_sc_dispatch = pl.kernel(
    _sc_dispatch_body,
    out_type=jax.ShapeDtypeStruct(((E + 1) * CAP, D), jnp.float32),
    mesh=_SC_MESH,
    scratch_types=[
        pltpu.VMEM((_TPW, D), jnp.float32),
        pltpu.VMEM((_TPW,), jnp.int32),
        pltpu.VMEM((_TPW,), jnp.int32),
        pltpu.SemaphoreType.DMA,
    ],
)


@jax.jit
def kernel(x, Wg, W1, W3, W2):
    logits = x @ Wg.T

    e0, e1, d, p0, p1, colsum, z = pl.pallas_call(
        _routing_kernel,
        out_shape=(
            jax.ShapeDtypeStruct((T, 1), jnp.int32),
            jax.ShapeDtypeStruct((T, 1), jnp.int32),
            jax.ShapeDtypeStruct((T, 1), jnp.float32),
            jax.ShapeDtypeStruct((T, 1), jnp.float32),
            jax.ShapeDtypeStruct((T, 1), jnp.float32),
            jax.ShapeDtypeStruct((1, E), jnp.float32),
            jax.ShapeDtypeStruct((1, 1), jnp.float32),
        ),
    )(logits)

    s_flat = jnp.concatenate([d, -d], axis=1).reshape(A, 1)
    e_flat = jnp.concatenate([e0, e1], axis=1).reshape(A, 1)

    rank, keep, dst = pl.pallas_call(
        _rank_kernel,
        grid=(A // _BI,),
        in_specs=[
            pl.BlockSpec((_BI, 1), lambda i: (i, 0)),
            pl.BlockSpec((_BI, 1), lambda i: (i, 0)),
            pl.BlockSpec((1, A), lambda i: (0, 0)),
            pl.BlockSpec((1, A), lambda i: (0, 0)),
        ],
        out_shape=(
            jax.ShapeDtypeStruct((A, 1), jnp.int32),
            jax.ShapeDtypeStruct((A, 1), jnp.int32),
            jax.ShapeDtypeStruct((A, 1), jnp.int32),
        ),
        out_specs=tuple(
            pl.BlockSpec((_BI, 1), lambda i: (i, 0)) for _ in range(3)),
    )(s_flat, e_flat, s_flat.reshape(1, A), e_flat.reshape(1, A))

    rank2 = rank.reshape(T, K)
    keep2 = keep.reshape(T, K)
    r0c, r1c = rank2[:, 0:1], rank2[:, 1:2]
    k0c, k1c = keep2[:, 0:1], keep2[:, 1:2]
    dst2 = dst.reshape(T, K)

    buf = _sc_dispatch(x, dst2[:, 0], dst2[:, 1])

    out, lb = pl.pallas_call(
        _expert_kernel,
        grid=(E, _FB),
        in_specs=[
            pl.BlockSpec((CAP, D), lambda e, fb: (e, 0)),
            pl.BlockSpec((1, _FBLK, D), lambda e, fb: (e, fb, 0)),
            pl.BlockSpec((1, _FBLK, D), lambda e, fb: (e, fb, 0)),
            pl.BlockSpec((1, D, _FBLK), lambda e, fb: (e, 0, fb)),
        ] + [pl.BlockSpec((T, 1), lambda e, fb: (0, 0))] * 8
          + [pl.BlockSpec((1, E), lambda e, fb: (0, 0))],
        out_shape=(
            jax.ShapeDtypeStruct((T, D), jnp.float32),
            jax.ShapeDtypeStruct((1, 1), jnp.float32),
        ),
        out_specs=(
            pl.BlockSpec((T, D), lambda e, fb: (0, 0)),
            pl.BlockSpec((1, 1), lambda e, fb: (0, 0)),
        ),
        scratch_shapes=[
            pltpu.VMEM((CAP, D), jnp.float32),
        ],
    )(buf, W1, W3, W2,
      e0, e1, r0c, r1c, k0c, k1c, p0, p1, colsum)

    return out, lb.reshape(()), z.reshape(())
